# baseline (speedup 1.0000x reference)
import functools

import jax
import jax.numpy as jnp
from jax import lax
from jax.experimental import pallas as pl
from jax.experimental.pallas import tpu as pltpu
from jax.experimental.pallas import tpu_sc as plsc

N_NODES = 10000
N_EDGES = 160000
F = 256
FH = 128
NC = 2
NS = 16
EPS = N_EDGES // NS
B = 80
NB = EPS // B
NBUF = 2
VPC = EPS // 16
CH = 200
NCH = N_NODES // CH
MM_ROWS = 1000


def _sc_body(x2, src_hbm, dst3_hbm, out3, srcf, sidx2d,
             r0, r1, acc, s0, s1, s4, s5):
    rows = [r0, r1]
    qsems = [[s0, s1], [s4, s5]]
    NSPL = 2
    HB = B // NSPL
    c = lax.axis_index("c")
    s = lax.axis_index("s")

    def fire(b, p):
        for q in range(NSPL):
            pltpu.async_copy(
                x2.at[srcf.at[pl.ds(b * B + q * HB, HB)]],
                rows[p].at[pl.ds(q * HB, HB)], qsems[p][q])

    def drain(p):
        for q in range(NSPL):
            pltpu.make_async_copy(
                x2.at[srcf.at[pl.ds(0, HB)]],
                rows[p].at[pl.ds(q * HB, HB)], qsems[p][q]).wait()

    zero16 = jnp.zeros((16,), jnp.float32)

    def zfill(i, carry):
        rows[0][i // (FH // 16), pl.ds((i % (FH // 16)) * 16, 16)] = zero16
        return carry

    lax.fori_loop(0, B * (FH // 16), zfill, 0)

    pltpu.async_copy(src_hbm.at[pl.ds(s * EPS, EPS)], srcf, qsems[1][0])
    pltpu.async_copy(dst3_hbm.at[s], sidx2d, qsems[1][0])

    nzc = N_NODES // B
    for k in range(pl.cdiv(nzc, NS)):
        j = s + NS * k

        @pl.when(j < nzc)
        def _():
            pltpu.async_copy(rows[0], acc.at[pl.ds(j * B, B)], qsems[0][0])

    pltpu.make_async_copy(src_hbm.at[pl.ds(0, EPS)], srcf, qsems[1][0]).wait()
    pltpu.make_async_copy(dst3_hbm.at[0], sidx2d, qsems[1][0]).wait()

    def tbody(j, carry):
        sv = srcf[pl.ds(j * 16, 16)]
        srcf[pl.ds(j * 16, 16)] = sv * 2 + c
        return carry

    lax.fori_loop(0, VPC, tbody, 0)

    fire(1, 1)

    for k in range(pl.cdiv(nzc, NS)):
        j = s + NS * k

        @pl.when(j < nzc)
        def _():
            pltpu.make_async_copy(rows[0], acc.at[pl.ds(0, B)], qsems[0][0]).wait()

    fire(0, 0)
    plsc.subcore_barrier()

    def round_body(i, carry):
        g = i * NBUF
        for p in range(NBUF):
            drain(p)
            pltpu.sync_copy(rows[p], acc.at[sidx2d.at[g + p]], add=True)
            fire(g + NBUF + p, p)
        return carry

    nr = (NB - NBUF - 1) // NBUF
    lax.fori_loop(0, nr, round_body, 0)
    for p in range(NBUF):
        drain(p)
        pltpu.sync_copy(rows[p], acc.at[sidx2d.at[nr * NBUF + p]], add=True)
        if p < NB - (nr + 1) * NBUF:
            fire((nr + 1) * NBUF + p, p)
    for p in range(NB - (nr + 1) * NBUF):
        drain(p)
        pltpu.sync_copy(rows[p], acc.at[sidx2d.at[(nr + 1) * NBUF + p]], add=True)

    plsc.subcore_barrier()

    for k in range(pl.cdiv(NCH, NS)):
        j = s + NS * k

        @pl.when(j < NCH)
        def _():
            pltpu.async_copy(
                acc.at[pl.ds(j * CH, CH)], out3.at[c, pl.ds(j * CH, CH)], qsems[0][0])

    for k in range(pl.cdiv(NCH, NS)):
        j = s + NS * k

        @pl.when(j < NCH)
        def _():
            pltpu.make_async_copy(
                acc.at[pl.ds(0, CH)], out3.at[c, pl.ds(0, CH)], qsems[0][0]).wait()


def _mm_body(a_ref, w_ref, b_ref, o_ref):
    o_ref[...] = (
        jnp.dot(a_ref[0], w_ref[0], preferred_element_type=jnp.float32)
        + jnp.dot(a_ref[1], w_ref[1], preferred_element_type=jnp.float32)
        + b_ref[...]
    )


@jax.jit
def kernel(x, edge_index, weight, bias):
    x2 = x.reshape(N_NODES * 2, FH)
    src = edge_index[0]
    dst3 = edge_index[1].reshape(NS, NB, B)

    mesh = plsc.VectorSubcoreMesh(core_axis_name="c", subcore_axis_name="s")
    agg3 = pl.kernel(
        _sc_body,
        out_type=jax.ShapeDtypeStruct((NC, N_NODES, FH), jnp.float32),
        mesh=mesh,
        scratch_types=[
            pltpu.VMEM((EPS,), jnp.int32),
            pltpu.VMEM((NB, B), jnp.int32),
            pltpu.VMEM((B, FH), jnp.float32),
            pltpu.VMEM((B, FH), jnp.float32),
            pltpu.VMEM_SHARED((N_NODES, FH), jnp.float32),
            pltpu.SemaphoreType.DMA,
            pltpu.SemaphoreType.DMA,
            pltpu.SemaphoreType.DMA,
            pltpu.SemaphoreType.DMA,
        ],
    )(x2, src, dst3)

    w3 = weight.reshape(NC, FH, F)
    out = pl.pallas_call(
        _mm_body,
        grid=(N_NODES // MM_ROWS,),
        in_specs=[
            pl.BlockSpec((NC, MM_ROWS, FH), lambda i: (0, i, 0)),
            pl.BlockSpec((NC, FH, F), lambda i: (0, 0, 0)),
            pl.BlockSpec((1, F), lambda i: (0, 0)),
        ],
        out_specs=pl.BlockSpec((MM_ROWS, F), lambda i: (i, 0)),
        out_shape=jax.ShapeDtypeStruct((N_NODES, F), jnp.float32),
    )(agg3, w3, bias.reshape(1, F))
    return out


# speedup vs baseline: 1.0167x; 1.0167x over previous
import functools

import jax
import jax.numpy as jnp
from jax import lax
from jax.experimental import pallas as pl
from jax.experimental.pallas import tpu as pltpu
from jax.experimental.pallas import tpu_sc as plsc

N_NODES = 10000
N_EDGES = 160000
F = 256
FH = 128
NC = 2
NS = 16
EPS = N_EDGES // NS
B = 80
NB = EPS // B
NBUF = 2
VPC = EPS // 16
CH = 200
NCH = N_NODES // CH
MM_ROWS = 1000


def _sc_body(x2, src_hbm, dst3_hbm, out3, srcf, sidx2d,
             r0, r1, acc, s0, s1, s4, s5):
    rows = [r0, r1]
    qsems = [[s0, s1], [s4, s5]]
    NSPL = 2
    HB = B // NSPL
    c = lax.axis_index("c")
    s = lax.axis_index("s")

    def fire(b, p):
        for q in range(NSPL):
            pltpu.async_copy(
                x2.at[srcf.at[pl.ds(b * B + q * HB, HB)]],
                rows[p].at[pl.ds(q * HB, HB)], qsems[p][q])

    def drain(p):
        for q in range(NSPL):
            pltpu.make_async_copy(
                x2.at[srcf.at[pl.ds(0, HB)]],
                rows[p].at[pl.ds(q * HB, HB)], qsems[p][q]).wait()

    pltpu.async_copy(src_hbm.at[pl.ds(s * EPS, EPS)], srcf, qsems[1][0])
    pltpu.async_copy(dst3_hbm.at[s], sidx2d, qsems[1][0])

    zero16 = jnp.zeros((16,), jnp.float32)

    def zfill(i, carry):
        rows[0][i // (FH // 16), pl.ds((i % (FH // 16)) * 16, 16)] = zero16
        return carry

    lax.fori_loop(0, B * (FH // 16), zfill, 0)

    nzc = N_NODES // B
    for k in range(pl.cdiv(nzc, NS)):
        j = s + NS * k

        @pl.when(j < nzc)
        def _():
            pltpu.async_copy(rows[0], acc.at[pl.ds(j * B, B)], qsems[0][0])

    pltpu.make_async_copy(src_hbm.at[pl.ds(0, EPS)], srcf, qsems[1][0]).wait()
    pltpu.make_async_copy(dst3_hbm.at[0], sidx2d, qsems[1][0]).wait()

    def tbody(j, carry):
        sv = srcf[pl.ds(j * 16, 16)]
        srcf[pl.ds(j * 16, 16)] = sv * 2 + c
        return carry

    pv = 2 * B // 16
    lax.fori_loop(0, pv, tbody, 0)
    fire(1, 1)
    lax.fori_loop(pv, VPC, tbody, 0)

    for k in range(pl.cdiv(nzc, NS)):
        j = s + NS * k

        @pl.when(j < nzc)
        def _():
            pltpu.make_async_copy(rows[0], acc.at[pl.ds(0, B)], qsems[0][0]).wait()

    fire(0, 0)
    plsc.subcore_barrier()

    def round_body(i, carry):
        g = i * NBUF
        for p in range(NBUF):
            drain(p)
            pltpu.sync_copy(rows[p], acc.at[sidx2d.at[g + p]], add=True)
            fire(g + NBUF + p, p)
        return carry

    nr = (NB - NBUF - 1) // NBUF
    lax.fori_loop(0, nr, round_body, 0)
    for p in range(NBUF):
        drain(p)
        pltpu.sync_copy(rows[p], acc.at[sidx2d.at[nr * NBUF + p]], add=True)
        if p < NB - (nr + 1) * NBUF:
            fire((nr + 1) * NBUF + p, p)
    for p in range(NB - (nr + 1) * NBUF):
        drain(p)
        pltpu.sync_copy(rows[p], acc.at[sidx2d.at[(nr + 1) * NBUF + p]], add=True)

    plsc.subcore_barrier()

    for k in range(pl.cdiv(NCH, NS)):
        j = s + NS * k

        @pl.when(j < NCH)
        def _():
            pltpu.async_copy(
                acc.at[pl.ds(j * CH, CH)], out3.at[c, pl.ds(j * CH, CH)], qsems[0][0])

    for k in range(pl.cdiv(NCH, NS)):
        j = s + NS * k

        @pl.when(j < NCH)
        def _():
            pltpu.make_async_copy(
                acc.at[pl.ds(0, CH)], out3.at[c, pl.ds(0, CH)], qsems[0][0]).wait()


def _mm_body(a_ref, w_ref, b_ref, o_ref):
    o_ref[...] = (
        jnp.dot(a_ref[0], w_ref[0], preferred_element_type=jnp.float32)
        + jnp.dot(a_ref[1], w_ref[1], preferred_element_type=jnp.float32)
        + b_ref[...]
    )


@jax.jit
def kernel(x, edge_index, weight, bias):
    x2 = x.reshape(N_NODES * 2, FH)
    edge_index = edge_index.astype(jnp.int32)
    src = edge_index[0]
    dst3 = edge_index[1].reshape(NS, NB, B)

    mesh = plsc.VectorSubcoreMesh(core_axis_name="c", subcore_axis_name="s")
    agg3 = pl.kernel(
        _sc_body,
        out_type=jax.ShapeDtypeStruct((NC, N_NODES, FH), jnp.float32),
        mesh=mesh,
        scratch_types=[
            pltpu.VMEM((EPS,), jnp.int32),
            pltpu.VMEM((NB, B), jnp.int32),
            pltpu.VMEM((B, FH), jnp.float32),
            pltpu.VMEM((B, FH), jnp.float32),
            pltpu.VMEM_SHARED((N_NODES, FH), jnp.float32),
            pltpu.SemaphoreType.DMA,
            pltpu.SemaphoreType.DMA,
            pltpu.SemaphoreType.DMA,
            pltpu.SemaphoreType.DMA,
        ],
    )(x2, src, dst3)

    w3 = weight.reshape(NC, FH, F)
    out = pl.pallas_call(
        _mm_body,
        grid=(N_NODES // MM_ROWS,),
        in_specs=[
            pl.BlockSpec((NC, MM_ROWS, FH), lambda i: (0, i, 0)),
            pl.BlockSpec((NC, FH, F), lambda i: (0, 0, 0)),
            pl.BlockSpec((1, F), lambda i: (0, 0)),
        ],
        out_specs=pl.BlockSpec((MM_ROWS, F), lambda i: (i, 0)),
        out_shape=jax.ShapeDtypeStruct((N_NODES, F), jnp.float32),
    )(agg3, w3, bias.reshape(1, F))
    return out


# speedup vs baseline: 1.0291x; 1.0123x over previous
import functools

import jax
import jax.numpy as jnp
from jax import lax
from jax.experimental import pallas as pl
from jax.experimental.pallas import tpu as pltpu
from jax.experimental.pallas import tpu_sc as plsc

N_NODES = 10000
N_EDGES = 160000
F = 256
FH = 128
NC = 2
NS = 16
EPS = N_EDGES // NS
B = 80
NB = EPS // B
NBUF = 2
VPC = EPS // 16
CH = 200
NCH = N_NODES // CH
MM_ROWS = 2000


def _sc_body(x2, src_hbm, dst3_hbm, out3, srcf, sidx2d,
             r0, r1, acc, s0, s1, s4, s5):
    rows = [r0, r1]
    qsems = [[s0, s1], [s4, s5]]
    NSPL = 2
    HB = B // NSPL
    c = lax.axis_index("c")
    s = lax.axis_index("s")

    def fire(b, p):
        for q in range(NSPL):
            pltpu.async_copy(
                x2.at[srcf.at[pl.ds(b * B + q * HB, HB)]],
                rows[p].at[pl.ds(q * HB, HB)], qsems[p][q])

    def drain(p):
        for q in range(NSPL):
            pltpu.make_async_copy(
                x2.at[srcf.at[pl.ds(0, HB)]],
                rows[p].at[pl.ds(q * HB, HB)], qsems[p][q]).wait()

    pltpu.async_copy(src_hbm.at[pl.ds(s * EPS, EPS)], srcf, qsems[1][0])
    pltpu.async_copy(dst3_hbm.at[s], sidx2d, qsems[1][0])

    zero16 = jnp.zeros((16,), jnp.float32)

    def zfill(i, carry):
        rows[0][i // (FH // 16), pl.ds((i % (FH // 16)) * 16, 16)] = zero16
        return carry

    lax.fori_loop(0, B * (FH // 16), zfill, 0)

    nzc = N_NODES // B
    for k in range(pl.cdiv(nzc, NS)):
        j = s + NS * k

        @pl.when(j < nzc)
        def _():
            pltpu.async_copy(rows[0], acc.at[pl.ds(j * B, B)], qsems[0][0])

    pltpu.make_async_copy(src_hbm.at[pl.ds(0, EPS)], srcf, qsems[1][0]).wait()
    pltpu.make_async_copy(dst3_hbm.at[0], sidx2d, qsems[1][0]).wait()

    def tbody(j, carry):
        sv = srcf[pl.ds(j * 16, 16)]
        srcf[pl.ds(j * 16, 16)] = sv * 2 + c
        return carry

    pv = 2 * B // 16
    lax.fori_loop(0, pv, tbody, 0)
    fire(1, 1)
    lax.fori_loop(pv, VPC, tbody, 0)

    for k in range(pl.cdiv(nzc, NS)):
        j = s + NS * k

        @pl.when(j < nzc)
        def _():
            pltpu.make_async_copy(rows[0], acc.at[pl.ds(0, B)], qsems[0][0]).wait()

    fire(0, 0)
    plsc.subcore_barrier()

    def round_body(i, carry):
        g = i * NBUF
        for p in range(NBUF):
            drain(p)
            pltpu.sync_copy(rows[p], acc.at[sidx2d.at[g + p]], add=True)
            fire(g + NBUF + p, p)
        return carry

    nr = (NB - NBUF - 1) // NBUF
    lax.fori_loop(0, nr, round_body, 0)
    for p in range(NBUF):
        drain(p)
        pltpu.sync_copy(rows[p], acc.at[sidx2d.at[nr * NBUF + p]], add=True)
        if p < NB - (nr + 1) * NBUF:
            fire((nr + 1) * NBUF + p, p)
    for p in range(NB - (nr + 1) * NBUF):
        drain(p)
        pltpu.sync_copy(rows[p], acc.at[sidx2d.at[(nr + 1) * NBUF + p]], add=True)

    plsc.subcore_barrier()

    for k in range(pl.cdiv(NCH, NS)):
        j = s + NS * k

        @pl.when(j < NCH)
        def _():
            pltpu.async_copy(
                acc.at[pl.ds(j * CH, CH)], out3.at[c, pl.ds(j * CH, CH)], qsems[0][0])

    for k in range(pl.cdiv(NCH, NS)):
        j = s + NS * k

        @pl.when(j < NCH)
        def _():
            pltpu.make_async_copy(
                acc.at[pl.ds(0, CH)], out3.at[c, pl.ds(0, CH)], qsems[0][0]).wait()


def _mm_body(a_ref, w_ref, b_ref, o_ref):
    o_ref[...] = (
        jnp.dot(a_ref[0], w_ref[0], preferred_element_type=jnp.float32)
        + jnp.dot(a_ref[1], w_ref[1], preferred_element_type=jnp.float32)
        + b_ref[...]
    )


@jax.jit
def kernel(x, edge_index, weight, bias):
    x2 = x.reshape(N_NODES * 2, FH)
    edge_index = edge_index.astype(jnp.int32)
    src = edge_index[0]
    dst3 = edge_index[1].reshape(NS, NB, B)

    mesh = plsc.VectorSubcoreMesh(core_axis_name="c", subcore_axis_name="s")
    agg3 = pl.kernel(
        _sc_body,
        out_type=jax.ShapeDtypeStruct((NC, N_NODES, FH), jnp.float32),
        mesh=mesh,
        scratch_types=[
            pltpu.VMEM((EPS,), jnp.int32),
            pltpu.VMEM((NB, B), jnp.int32),
            pltpu.VMEM((B, FH), jnp.float32),
            pltpu.VMEM((B, FH), jnp.float32),
            pltpu.VMEM_SHARED((N_NODES, FH), jnp.float32),
            pltpu.SemaphoreType.DMA,
            pltpu.SemaphoreType.DMA,
            pltpu.SemaphoreType.DMA,
            pltpu.SemaphoreType.DMA,
        ],
    )(x2, src, dst3)

    w3 = weight.reshape(NC, FH, F)
    out = pl.pallas_call(
        _mm_body,
        grid=(N_NODES // MM_ROWS,),
        in_specs=[
            pl.BlockSpec((NC, MM_ROWS, FH), lambda i: (0, i, 0)),
            pl.BlockSpec((NC, FH, F), lambda i: (0, 0, 0)),
            pl.BlockSpec((1, F), lambda i: (0, 0)),
        ],
        out_specs=pl.BlockSpec((MM_ROWS, F), lambda i: (i, 0)),
        out_shape=jax.ShapeDtypeStruct((N_NODES, F), jnp.float32),
    )(agg3, w3, bias.reshape(1, F))
    return out


# speedup vs baseline: 1.0324x; 1.0032x over previous
import jax
import jax.numpy as jnp
from jax import lax
from jax.experimental import pallas as pl
from jax.experimental.pallas import tpu as pltpu
from jax.experimental.pallas import tpu_sc as plsc

N_NODES = 10000
N_EDGES = 160000
F = 256
FH = 128
NC = 2
NS = 16
EPS = N_EDGES // NS
B = 80
NB = EPS // B
NBUF = 2
VPC = EPS // 16
CH = 200
NCH = N_NODES // CH
MM_ROWS = 2000


def _sc_body(x2, src_hbm, dst3_hbm, out3, srcf, sidx2d,
             r0, r1, acc, s0, s1, s4, s5):
    rows = [r0, r1]
    qsems = [[s0, s1], [s4, s5]]
    NSPL = 2
    HB = B // NSPL
    c = lax.axis_index("c")
    s = lax.axis_index("s")

    def fire(b, p):
        for q in range(NSPL):
            pltpu.async_copy(
                x2.at[srcf.at[pl.ds(b * B + q * HB, HB)]],
                rows[p].at[pl.ds(q * HB, HB)], qsems[p][q])

    def drain(p):
        for q in range(NSPL):
            pltpu.make_async_copy(
                x2.at[srcf.at[pl.ds(0, HB)]],
                rows[p].at[pl.ds(q * HB, HB)], qsems[p][q]).wait()

    pltpu.async_copy(src_hbm.at[pl.ds(s * EPS, EPS)], srcf, qsems[1][0])
    pltpu.async_copy(dst3_hbm.at[s], sidx2d, qsems[1][0])

    zero16 = jnp.zeros((16,), jnp.float32)

    def zfill(i, carry):
        rows[0][i // (FH // 16), pl.ds((i % (FH // 16)) * 16, 16)] = zero16
        return carry

    lax.fori_loop(0, B * (FH // 16), zfill, 0)

    nzc = N_NODES // B
    for k in range(pl.cdiv(nzc, NS)):
        j = s + NS * k

        @pl.when(j < nzc)
        def _():
            pltpu.async_copy(rows[0], acc.at[pl.ds(j * B, B)], qsems[0][0])

    pltpu.make_async_copy(src_hbm.at[pl.ds(0, EPS)], srcf, qsems[1][0]).wait()
    pltpu.make_async_copy(dst3_hbm.at[0], sidx2d, qsems[1][0]).wait()

    def tbody(j, carry):
        sv = srcf[pl.ds(j * 16, 16)]
        srcf[pl.ds(j * 16, 16)] = sv * 2 + c
        return carry

    pv = 2 * B // 16
    lax.fori_loop(0, pv, tbody, 0)
    fire(1, 1)
    lax.fori_loop(pv, VPC, tbody, 0)

    for k in range(pl.cdiv(nzc, NS)):
        j = s + NS * k

        @pl.when(j < nzc)
        def _():
            pltpu.make_async_copy(rows[0], acc.at[pl.ds(0, B)], qsems[0][0]).wait()

    fire(0, 0)
    plsc.subcore_barrier()

    def round_body(i, carry):
        g = i * NBUF
        for p in range(NBUF):
            drain(p)
            pltpu.sync_copy(rows[p], acc.at[sidx2d.at[g + p]], add=True)
            fire(g + NBUF + p, p)
        return carry

    nr = (NB - NBUF - 1) // NBUF
    lax.fori_loop(0, nr, round_body, 0)
    for p in range(NBUF):
        drain(p)
        pltpu.sync_copy(rows[p], acc.at[sidx2d.at[nr * NBUF + p]], add=True)
        if p < NB - (nr + 1) * NBUF:
            fire((nr + 1) * NBUF + p, p)
    for p in range(NB - (nr + 1) * NBUF):
        drain(p)
        pltpu.sync_copy(rows[p], acc.at[sidx2d.at[(nr + 1) * NBUF + p]], add=True)

    plsc.subcore_barrier()

    for k in range(pl.cdiv(NCH, NS)):
        j = s + NS * k

        @pl.when(j < NCH)
        def _():
            pltpu.async_copy(
                acc.at[pl.ds(j * CH, CH)], out3.at[c, pl.ds(j * CH, CH)], qsems[0][0])

    for k in range(pl.cdiv(NCH, NS)):
        j = s + NS * k

        @pl.when(j < NCH)
        def _():
            pltpu.make_async_copy(
                acc.at[pl.ds(0, CH)], out3.at[c, pl.ds(0, CH)], qsems[0][0]).wait()


def _mm_body(a_ref, w_ref, b_ref, o_ref):
    o_ref[...] = (
        jnp.dot(a_ref[0], w_ref[0], preferred_element_type=jnp.float32)
        + jnp.dot(a_ref[1], w_ref[1], preferred_element_type=jnp.float32)
        + b_ref[...]
    )


@jax.jit
def kernel(x, edge_index, weight, bias):
    x2 = x.reshape(N_NODES * 2, FH)
    edge_index = edge_index.astype(jnp.int32)
    src = edge_index[0]
    dst3 = edge_index[1].reshape(NS, NB, B)

    mesh = plsc.VectorSubcoreMesh(core_axis_name="c", subcore_axis_name="s")
    agg3 = pl.kernel(
        _sc_body,
        out_type=jax.ShapeDtypeStruct((NC, N_NODES, FH), jnp.float32),
        mesh=mesh,
        scratch_types=[
            pltpu.VMEM((EPS,), jnp.int32),
            pltpu.VMEM((NB, B), jnp.int32),
            pltpu.VMEM((B, FH), jnp.float32),
            pltpu.VMEM((B, FH), jnp.float32),
            pltpu.VMEM_SHARED((N_NODES, FH), jnp.float32),
            pltpu.SemaphoreType.DMA,
            pltpu.SemaphoreType.DMA,
            pltpu.SemaphoreType.DMA,
            pltpu.SemaphoreType.DMA,
        ],
    )(x2, src, dst3)

    w3 = weight.reshape(NC, FH, F)
    out = pl.pallas_call(
        _mm_body,
        grid=(N_NODES // MM_ROWS,),
        in_specs=[
            pl.BlockSpec((NC, MM_ROWS, FH), lambda i: (0, i, 0)),
            pl.BlockSpec((NC, FH, F), lambda i: (0, 0, 0)),
            pl.BlockSpec((1, F), lambda i: (0, 0)),
        ],
        out_specs=pl.BlockSpec((MM_ROWS, F), lambda i: (i, 0)),
        out_shape=jax.ShapeDtypeStruct((N_NODES, F), jnp.float32),
    )(agg3, w3, bias.reshape(1, F))
    return out
